# Initial kernel scaffold; baseline (speedup 1.0000x reference)
import functools

import jax
import jax.numpy as jnp
from jax import lax
from jax.experimental import pallas as pl
from jax.experimental.pallas import tpu as pltpu
from jax.experimental.pallas import tpu_sc as plsc

N_NODES = 10000
D = 128
DE = 16

NC = 2
NS = 16
NW = NC * NS

B = 64
ROWS_PER_TILE = N_NODES // NS
ROW_CHUNK = 125

_HI = jax.lax.Precision.HIGHEST


def _lrelu(x):
    return jnp.where(x > 0, x, 0.2 * x)


def _tc_node_transform(x, w_src_cat, b_src_cat, w_dst_cat, b_dst_cat,
                       w_wn, b_wn):
    n = x.shape[0]
    rn = 2000
    grid = n // rn

    def body(x_ref, ws_ref, bs_ref, wd_ref, bd_ref, wn_ref, bn_ref,
             tsrc_ref, tdst_ref, own_ref):
        x_blk = x_ref[...]
        tsrc_ref[...] = jnp.dot(x_blk, ws_ref[...], precision=_HI) + bs_ref[...]
        tdst_ref[...] = jnp.dot(x_blk, wd_ref[...], precision=_HI) + bd_ref[...]
        own_ref[...] = jnp.dot(x_blk, wn_ref[...], precision=_HI) + bn_ref[...]

    full = lambda shape: pl.BlockSpec(shape, lambda i: (0,) * len(shape))
    return pl.pallas_call(
        body,
        grid=(grid,),
        in_specs=[
            pl.BlockSpec((rn, D), lambda i: (i, 0)),
            full((D, 2 * D)), full((1, 2 * D)),
            full((D, 2 * D)), full((1, 2 * D)),
            full((D, D)), full((1, D)),
        ],
        out_specs=[
            pl.BlockSpec((rn, 2 * D), lambda i: (i, 0)),
            pl.BlockSpec((rn, 2 * D), lambda i: (i, 0)),
            pl.BlockSpec((rn, D), lambda i: (i, 0)),
        ],
        out_shape=[
            jax.ShapeDtypeStruct((n, 2 * D), jnp.float32),
            jax.ShapeDtypeStruct((n, 2 * D), jnp.float32),
            jax.ShapeDtypeStruct((n, D), jnp.float32),
        ],
    )(x, w_src_cat, b_src_cat, w_dst_cat, b_dst_cat, w_wn, b_wn)


def _tc_edge_transform(ef, w_edg_cat, b_edg_cat):
    epad = ef.shape[0]
    re = 5024
    grid = epad // re

    def body(ef_ref, w_ref, b_ref, out_ref):
        out_ref[...] = (
            jnp.dot(ef_ref[...], w_ref[...], precision=_HI) + b_ref[...])

    return pl.pallas_call(
        body,
        grid=(grid,),
        in_specs=[
            pl.BlockSpec((re, DE), lambda i: (i, 0)),
            pl.BlockSpec((DE, 2 * D), lambda i: (0, 0)),
            pl.BlockSpec((1, 2 * D), lambda i: (0, 0)),
        ],
        out_specs=pl.BlockSpec((re, 2 * D), lambda i: (i, 0)),
        out_shape=jax.ShapeDtypeStruct((epad, 2 * D), jnp.float32),
    )(ef, w_edg_cat, b_edg_cat)


def _sc_edge_pass(tsrc, tdst, tedg, upad, vpad, wdot, b0p, n_edges):
    epad = upad.shape[0]
    ept = epad // NW
    nb = ept // B
    mesh = plsc.VectorSubcoreMesh(core_axis_name="c", subcore_axis_name="s")

    @functools.partial(
        pl.kernel,
        mesh=mesh,
        out_type=[
            jax.ShapeDtypeStruct((NW, N_NODES), jnp.float32),
            jax.ShapeDtypeStruct((NC, N_NODES, D), jnp.float32),
        ],
        scratch_types=[
            pltpu.VMEM((B,), jnp.int32),
            pltpu.VMEM((B,), jnp.int32),
            pltpu.VMEM((B, 2 * D), jnp.float32),
            pltpu.VMEM((B, 2 * D), jnp.float32),
            pltpu.VMEM((B, 2 * D), jnp.float32),
            pltpu.VMEM((B, D), jnp.float32),
            pltpu.VMEM((B,), jnp.float32),
            pltpu.VMEM((N_NODES,), jnp.float32),
            pltpu.VMEM((D,), jnp.float32),
            pltpu.VMEM((8,), jnp.float32),
            pltpu.VMEM((ROW_CHUNK, D), jnp.float32),
            pltpu.VMEM_SHARED((N_NODES, D), jnp.float32),
            pltpu.SemaphoreType.DMA,
            pltpu.SemaphoreType.DMA,
        ],
    )
    def sc_kernel(tsrc_h, tdst_h, tedg_h, u_h, v_h, w_h, b0_h,
                  exsum_h, pmsg_h,
                  u_v, v_v, g_s, g_d, g_e, msg_v, ex_v, tbl, w_v, b0_v,
                  zbuf, shacc, sem1, sem2):
        c = lax.axis_index("c")
        s = lax.axis_index("s")
        wid = c * NS + s

        pltpu.sync_copy(w_h, w_v)
        pltpu.sync_copy(b0_h, b0_v)
        wch = [w_v[pl.ds(i * 16, 16)] for i in range(8)]
        b0s = b0_v[0]
        zero16 = jnp.zeros((16,), jnp.float32)

        def zt(i, carry):
            tbl[pl.ds(i * 16, 16)] = zero16
            return carry
        lax.fori_loop(0, N_NODES // 16, zt, 0)

        def zz(i, carry):
            for ci in range(8):
                zbuf[i, pl.ds(ci * 16, 16)] = zero16
            return carry
        lax.fori_loop(0, ROW_CHUNK, zz, 0)

        def zs(j, carry):
            pltpu.sync_copy(
                zbuf, shacc.at[pl.ds(s * ROWS_PER_TILE + j * ROW_CHUNK,
                                     ROW_CHUNK)])
            return carry
        lax.fori_loop(0, ROWS_PER_TILE // ROW_CHUNK, zs, 0)
        plsc.subcore_barrier()

        ebase0 = wid * ept

        def batch(ib, carry):
            base = ebase0 + ib * B
            pltpu.sync_copy(u_h.at[pl.ds(base, B)], u_v)
            pltpu.sync_copy(v_h.at[pl.ds(base, B)], v_v)
            cp1 = pltpu.async_copy(tsrc_h.at[u_v], g_s, sem1)
            cp2 = pltpu.async_copy(tdst_h.at[v_v], g_d, sem2)
            pltpu.sync_copy(tedg_h.at[pl.ds(base, B)], g_e)
            cp1.wait()
            cp2.wait()

            def att(e, carry2):
                acc0 = zero16
                acc1 = zero16
                for ci in range(8):
                    sl = pl.ds(ci * 16, 16)
                    q = g_s[e, sl] + g_d[e, sl] + g_e[e, sl]
                    q = _lrelu(q)
                    if ci % 2 == 0:
                        acc0 = acc0 + wch[ci] * q
                    else:
                        acc1 = acc1 + wch[ci] * q
                ex_v[e] = jnp.sum(acc0 + acc1) + b0s
                return carry2
            lax.fori_loop(0, B, att, 0, unroll=2)

            for g in range(B // 16):
                sl = pl.ds(g * 16, 16)
                lgv = ex_v[sl]
                eid = base + g * 16 + lax.iota(jnp.int32, 16)
                ex = jnp.where(eid < n_edges, jnp.exp(lgv), 0.0)
                ex_v[sl] = ex
                plsc.addupdate_scatter(tbl, [v_v[sl]], ex)

            def msg(e, carry2):
                exv = jnp.full((16,), ex_v[e])
                for ci in range(8):
                    sl = pl.ds(D + ci * 16, 16)
                    t = g_s[e, sl] + g_d[e, sl] + g_e[e, sl]
                    msg_v[e, pl.ds(ci * 16, 16)] = _lrelu(t) * exv
                return carry2
            lax.fori_loop(0, B, msg, 0, unroll=2)

            pltpu.sync_copy(msg_v, shacc.at[v_v], add=True)
            return carry
        lax.fori_loop(0, nb, batch, 0)

        pltpu.sync_copy(tbl, exsum_h.at[wid])
        plsc.subcore_barrier()

        def wo(j, carry):
            r0 = s * ROWS_PER_TILE + j * ROW_CHUNK
            pltpu.sync_copy(shacc.at[pl.ds(r0, ROW_CHUNK)], zbuf)
            pltpu.sync_copy(zbuf, pmsg_h.at[c, pl.ds(r0, ROW_CHUNK)])
            return carry
        lax.fori_loop(0, ROWS_PER_TILE // ROW_CHUNK, wo, 0)

    return sc_kernel(tsrc, tdst, tedg, upad, vpad, wdot, b0p)


def _tc_combine(pmsg, exsum, wn):
    n = wn.shape[0]
    rn = 2000
    grid = n // rn

    def body(p_ref, s_ref, wn_ref, out_ref):
        p = p_ref[0] + p_ref[1]
        ssum = jnp.sum(s_ref[...], axis=0)
        r = jnp.where(ssum > 0, 1.0 / ssum, 0.0)
        out_ref[...] = _lrelu(p * r[:, None] + wn_ref[...])

    return pl.pallas_call(
        body,
        grid=(grid,),
        in_specs=[
            pl.BlockSpec((NC, rn, D), lambda i: (0, i, 0)),
            pl.BlockSpec((NW, rn), lambda i: (0, i)),
            pl.BlockSpec((rn, D), lambda i: (i, 0)),
        ],
        out_specs=pl.BlockSpec((rn, D), lambda i: (i, 0)),
        out_shape=jax.ShapeDtypeStruct((n, D), jnp.float32),
    )(pmsg, exsum, wn)


def kernel(node_feats, edge_feats, edge_index,
           W_attn_src, b_attn_src, W_attn_dst, b_attn_dst,
           W_attn_edg, b_attn_edg, W_attn_dot, b_attn_dot,
           W_msg_src, b_msg_src, W_msg_dst, b_msg_dst,
           W_msg_edg, b_msg_edg, W_wgt_n, b_wgt_n):
    n = node_feats.shape[0]
    e = edge_feats.shape[0]
    x = node_feats.reshape(n, D)

    ept = ((e + NW * B - 1) // (NW * B)) * B
    epad = ept * NW
    pad = epad - e
    upad = jnp.pad(edge_index[0], (0, pad))
    vpad = jnp.pad(edge_index[1], (0, pad))
    efpad = jnp.pad(edge_feats, ((0, pad), (0, 0)))

    w_src_cat = jnp.concatenate([W_attn_src, W_msg_src], axis=0).T
    b_src_cat = jnp.concatenate([b_attn_src, b_msg_src]).reshape(1, 2 * D)
    w_dst_cat = jnp.concatenate([W_attn_dst, W_msg_dst], axis=0).T
    b_dst_cat = jnp.concatenate([b_attn_dst, b_msg_dst]).reshape(1, 2 * D)
    w_edg_cat = jnp.concatenate([W_attn_edg, W_msg_edg], axis=0).T
    b_edg_cat = jnp.concatenate([b_attn_edg, b_msg_edg]).reshape(1, 2 * D)

    tsrc, tdst, wn = _tc_node_transform(
        x, w_src_cat, b_src_cat, w_dst_cat, b_dst_cat,
        W_wgt_n.T, b_wgt_n.reshape(1, D))
    tedg = _tc_edge_transform(efpad, w_edg_cat, b_edg_cat)

    wdot = W_attn_dot.reshape(D)
    b0p = jnp.pad(b_attn_dot, (0, 7))

    exsum, pmsg = _sc_edge_pass(tsrc, tdst, tedg, upad, vpad, wdot, b0p, e)
    out = _tc_combine(pmsg, exsum, wn)
    return out.reshape(n, 1, D)


# speedup vs baseline: 3.0980x; 3.0980x over previous
import functools

import jax
import jax.numpy as jnp
from jax import lax
from jax.experimental import pallas as pl
from jax.experimental.pallas import tpu as pltpu
from jax.experimental.pallas import tpu_sc as plsc

N_NODES = 10000
D = 128
DH = 32
NPH = D // DH
DE = 16

NC = 2
NS = 16
NW = NC * NS

B = 128
N_PAD = 10240
ROWS_PER_TILE = N_PAD // NS
ROW_CHUNK = 128

_HI = jax.lax.Precision.HIGHEST


def _lrelu(x):
    return jnp.where(x > 0, x, 0.2 * x)


def _tc_node_transform(x, w_att_cat, b_att_cat, w_msg_cat, b_msg_cat,
                       w_wn, b_wn):
    n = x.shape[0]
    rn = 2000
    grid = n // rn

    def body(x_ref, wa_ref, ba_ref, wm_ref, bm_ref, wn_ref, bn_ref,
             a_ref, b_ref, *rest):
        ms_refs = rest[0:NPH]
        md_refs = rest[NPH:2 * NPH]
        own_ref = rest[2 * NPH]
        x_blk = x_ref[...]
        att = jnp.dot(x_blk, wa_ref[...], precision=_HI) + ba_ref[...]
        a_ref[...] = att[:, :D]
        b_ref[...] = att[:, D:]
        msg = jnp.dot(x_blk, wm_ref[...], precision=_HI) + bm_ref[...]
        for q in range(NPH):
            ms_refs[q][...] = msg[:, q * DH:(q + 1) * DH]
            md_refs[q][...] = msg[:, D + q * DH:D + (q + 1) * DH]
        own_ref[...] = jnp.dot(x_blk, wn_ref[...], precision=_HI) + bn_ref[...]

    full = lambda shape: pl.BlockSpec(shape, lambda i: (0,) * len(shape))
    half = pl.BlockSpec((rn, DH), lambda i: (i, 0))
    return pl.pallas_call(
        body,
        grid=(grid,),
        in_specs=[
            pl.BlockSpec((rn, D), lambda i: (i, 0)),
            full((D, 2 * D)), full((1, 2 * D)),
            full((D, 2 * D)), full((1, 2 * D)),
            full((D, D)), full((1, D)),
        ],
        out_specs=(
            [pl.BlockSpec((rn, D), lambda i: (i, 0))] * 2
            + [half] * (2 * NPH)
            + [pl.BlockSpec((rn, D), lambda i: (i, 0))]),
        out_shape=(
            [jax.ShapeDtypeStruct((n, D), jnp.float32)] * 2
            + [jax.ShapeDtypeStruct((n, DH), jnp.float32)] * (2 * NPH)
            + [jax.ShapeDtypeStruct((n, D), jnp.float32)]),
    )(x, w_att_cat, b_att_cat, w_msg_cat, b_msg_cat, w_wn, b_wn)


def _tc_edge_transform(ef, w_edg_cat, b_edg_cat):
    epad = ef.shape[0]
    re = 5056
    grid = epad // re

    def body(ef_ref, w_ref, b_ref, ea_ref, *em_refs):
        e_blk = jnp.dot(ef_ref[...], w_ref[...], precision=_HI) + b_ref[...]
        ea_ref[...] = e_blk[:, :D]
        for q in range(NPH):
            em_refs[q][...] = e_blk[:, D + q * DH:D + (q + 1) * DH]

    return pl.pallas_call(
        body,
        grid=(grid,),
        in_specs=[
            pl.BlockSpec((re, DE), lambda i: (i, 0)),
            pl.BlockSpec((DE, 2 * D), lambda i: (0, 0)),
            pl.BlockSpec((1, 2 * D), lambda i: (0, 0)),
        ],
        out_specs=(
            [pl.BlockSpec((re, D), lambda i: (i, 0))]
            + [pl.BlockSpec((re, DH), lambda i: (i, 0))] * NPH),
        out_shape=(
            [jax.ShapeDtypeStruct((epad, D), jnp.float32)]
            + [jax.ShapeDtypeStruct((epad, DH), jnp.float32)] * NPH),
    )(ef, w_edg_cat, b_edg_cat)


def _sc_edge_pass(ta, tb, ms_q, md_q, ea, em_q,
                  upad, vpad, wdot, b0p, n_edges):
    epad = upad.shape[0]
    ept = epad // NW
    nb = ept // B
    mesh = plsc.VectorSubcoreMesh(core_axis_name="c", subcore_axis_name="s")

    @functools.partial(
        pl.kernel,
        mesh=mesh,
        compiler_params=pltpu.CompilerParams(
            needs_layout_passes=False, use_tc_tiling_on_sc=False),
        out_type=[
            jax.ShapeDtypeStruct((NW, 1, N_NODES), jnp.float32),
            jax.ShapeDtypeStruct((NC, NPH, N_PAD, DH), jnp.float32),
        ],
        scratch_types=[
            pltpu.VMEM((B,), jnp.int32),
            pltpu.VMEM((B,), jnp.int32),
            pltpu.VMEM((B, D), jnp.float32),
            pltpu.VMEM((B, D), jnp.float32),
            pltpu.VMEM((B, D), jnp.float32),
            pltpu.VMEM((B, DH), jnp.float32),
            pltpu.VMEM((B, DH), jnp.float32),
            pltpu.VMEM((B, DH), jnp.float32),
            pltpu.VMEM((B, DH), jnp.float32),
            pltpu.VMEM((ept,), jnp.float32),
            pltpu.VMEM((N_NODES,), jnp.float32),
            pltpu.VMEM((D,), jnp.float32),
            pltpu.VMEM((16,), jnp.float32),
            pltpu.VMEM((ROW_CHUNK, DH), jnp.float32),
            pltpu.VMEM_SHARED((N_PAD, DH), jnp.float32),
            pltpu.SemaphoreType.DMA,
            pltpu.SemaphoreType.DMA,
        ],
    )
    def sc_kernel(ta_h, tb_h,
                  ms0_h, ms1_h, ms2_h, ms3_h, md0_h, md1_h, md2_h, md3_h,
                  ea_h, em0_h, em1_h, em2_h, em3_h,
                  u_h, v_h, w_h, b0_h,
                  exsum_h, pmsg_h,
                  u_v, v_v, g_a, g_b, g_e, g_ms, g_md, g_em, msg_v,
                  ex_all, tbl, w_v, b0_v, zbuf, shacc, sem1, sem2):
        ms_hs = [ms0_h, ms1_h, ms2_h, ms3_h]
        md_hs = [md0_h, md1_h, md2_h, md3_h]
        em_hs = [em0_h, em1_h, em2_h, em3_h]
        c = lax.axis_index("c")
        s = lax.axis_index("s")
        wid = c * NS + s

        pltpu.sync_copy(w_h, w_v)
        pltpu.sync_copy(b0_h, b0_v)
        wch = [w_v[pl.ds(i * 16, 16)] for i in range(8)]
        b0s = b0_v[...][0]
        zero16 = jnp.zeros((16,), jnp.float32)
        iota16 = lax.iota(jnp.int32, 16)

        def zt(i, carry):
            tbl[pl.ds(i * 16, 16)] = zero16
            return carry
        lax.fori_loop(0, N_NODES // 16, zt, 0)

        def zz(i, carry):
            for ci in range(DH // 16):
                zbuf[i, pl.ds(ci * 16, 16)] = zero16
            return carry
        lax.fori_loop(0, ROW_CHUNK, zz, 0)

        def zero_shacc(j, carry):
            pltpu.sync_copy(
                zbuf, shacc.at[pl.ds(s * ROWS_PER_TILE + j * ROW_CHUNK,
                                     ROW_CHUNK)])
            return carry

        ebase0 = wid * ept

        def batch_att(ib, carry):
            base = ebase0 + ib * B
            pltpu.sync_copy(u_h.at[pl.ds(base, B)], u_v)
            pltpu.sync_copy(v_h.at[pl.ds(base, B)], v_v)
            cp1 = pltpu.async_copy(ta_h.at[u_v], g_a, sem1)
            cp2 = pltpu.async_copy(tb_h.at[v_v], g_b, sem2)
            pltpu.sync_copy(ea_h.at[pl.ds(base, B)], g_e)
            cp1.wait()
            cp2.wait()

            def att_grp(g, carry2):
                e0 = g * 16
                lgv = zero16
                for j in range(16):
                    e = e0 + j
                    acc0 = zero16
                    acc1 = zero16
                    for ci in range(8):
                        sl = pl.ds(ci * 16, 16)
                        q = g_a[e, sl] + g_b[e, sl] + g_e[e, sl]
                        q = _lrelu(q)
                        if ci % 2 == 0:
                            acc0 = acc0 + wch[ci] * q
                        else:
                            acc1 = acc1 + wch[ci] * q
                    lg = jnp.sum(acc0 + acc1)
                    lgv = jnp.where(iota16 == j, lg, lgv)
                eid = base + e0 + iota16
                ex = jnp.where(eid < n_edges, jnp.exp(lgv + b0s), 0.0)
                sl = pl.ds(e0, 16)
                ex_all[pl.ds(ib * B + e0, 16)] = ex
                plsc.addupdate_scatter(tbl, [v_v[sl]], ex)
                return carry2
            lax.fori_loop(0, B // 16, att_grp, 0)
            return carry
        lax.fori_loop(0, nb, batch_att, 0)
        pltpu.sync_copy(tbl, exsum_h.at[wid, 0])

        for ph in range(NPH):
            ms_h, md_h, em_h = ms_hs[ph], md_hs[ph], em_hs[ph]
            lax.fori_loop(0, ROWS_PER_TILE // ROW_CHUNK, zero_shacc, 0)
            plsc.subcore_barrier()

            def batch_msg(ib, carry):
                base = ebase0 + ib * B
                pltpu.sync_copy(u_h.at[pl.ds(base, B)], u_v)
                pltpu.sync_copy(v_h.at[pl.ds(base, B)], v_v)
                cp1 = pltpu.async_copy(ms_h.at[u_v], g_ms, sem1)
                cp2 = pltpu.async_copy(md_h.at[v_v], g_md, sem2)
                pltpu.sync_copy(em_h.at[pl.ds(base, B)], g_em)
                cp1.wait()
                cp2.wait()

                def msg_grp(g, carry2):
                    e0 = g * 16
                    exg = ex_all[pl.ds(ib * B + e0, 16)]
                    for j in range(16):
                        e = e0 + j
                        exv = jnp.full((16,), exg[j])
                        for ci in range(DH // 16):
                            sl = pl.ds(ci * 16, 16)
                            t = g_ms[e, sl] + g_md[e, sl] + g_em[e, sl]
                            msg_v[e, sl] = _lrelu(t) * exv
                    return carry2
                lax.fori_loop(0, B // 16, msg_grp, 0)

                pltpu.sync_copy(msg_v, shacc.at[v_v], add=True)
                return carry
            lax.fori_loop(0, nb, batch_msg, 0)
            plsc.subcore_barrier()

            def wo(j, carry):
                r0 = s * ROWS_PER_TILE + j * ROW_CHUNK
                pltpu.sync_copy(shacc.at[pl.ds(r0, ROW_CHUNK)], zbuf)
                pltpu.sync_copy(zbuf, pmsg_h.at[c, ph, pl.ds(r0, ROW_CHUNK)])
                return carry
            lax.fori_loop(0, ROWS_PER_TILE // ROW_CHUNK, wo, 0)
            plsc.subcore_barrier()
            lax.fori_loop(0, ROW_CHUNK, zz, 0)

    return sc_kernel(ta, tb, *ms_q, *md_q, ea, *em_q,
                     upad, vpad, wdot, b0p)


def _tc_combine(pmsg, exsum, wn):
    n = wn.shape[0]
    rn = 2000
    grid = n // rn

    def body(p_ref, s_ref, wn_ref, out_ref):
        p = jnp.concatenate(
            [p_ref[0, q] + p_ref[1, q] for q in range(NPH)], axis=1)
        ssum = jnp.sum(s_ref[...], axis=1)
        r = jnp.where(ssum > 0, 1.0 / ssum, 0.0)
        out_ref[...] = _lrelu(p * r[:, None] + wn_ref[...])

    return pl.pallas_call(
        body,
        grid=(grid,),
        in_specs=[
            pl.BlockSpec((NC, NPH, rn, DH), lambda i: (0, 0, i, 0)),
            pl.BlockSpec((rn, NW), lambda i: (i, 0)),
            pl.BlockSpec((rn, D), lambda i: (i, 0)),
        ],
        out_specs=pl.BlockSpec((rn, D), lambda i: (i, 0)),
        out_shape=jax.ShapeDtypeStruct((n, D), jnp.float32),
    )(pmsg, exsum, wn)


def kernel(node_feats, edge_feats, edge_index,
           W_attn_src, b_attn_src, W_attn_dst, b_attn_dst,
           W_attn_edg, b_attn_edg, W_attn_dot, b_attn_dot,
           W_msg_src, b_msg_src, W_msg_dst, b_msg_dst,
           W_msg_edg, b_msg_edg, W_wgt_n, b_wgt_n):
    n = node_feats.shape[0]
    e = edge_feats.shape[0]
    x = node_feats.reshape(n, D)

    ept = ((e + NW * B - 1) // (NW * B)) * B
    epad = ept * NW
    pad = epad - e
    upad = jnp.pad(edge_index[0], (0, pad))
    vpad = jnp.pad(edge_index[1], (0, pad))
    efpad = jnp.pad(edge_feats, ((0, pad), (0, 0)))

    w_att_cat = jnp.concatenate([W_attn_src, W_attn_dst], axis=0).T
    b_att_cat = jnp.concatenate([b_attn_src, b_attn_dst]).reshape(1, 2 * D)
    w_msg_cat = jnp.concatenate([W_msg_src, W_msg_dst], axis=0).T
    b_msg_cat = jnp.concatenate([b_msg_src, b_msg_dst]).reshape(1, 2 * D)
    w_edg_cat = jnp.concatenate([W_attn_edg, W_msg_edg], axis=0).T
    b_edg_cat = jnp.concatenate([b_attn_edg, b_msg_edg]).reshape(1, 2 * D)

    outs = _tc_node_transform(
        x, w_att_cat, b_att_cat, w_msg_cat, b_msg_cat,
        W_wgt_n.T, b_wgt_n.reshape(1, D))
    ta, tb = outs[0], outs[1]
    ms_q = outs[2:2 + NPH]
    md_q = outs[2 + NPH:2 + 2 * NPH]
    wn = outs[2 + 2 * NPH]
    ea, *em_q = _tc_edge_transform(efpad, w_edg_cat, b_edg_cat)

    wdot = W_attn_dot.reshape(D)
    b0p = jnp.pad(b_attn_dot, (0, 15))

    exsum, pmsg = _sc_edge_pass(ta, tb, ms_q, md_q, ea, em_q,
                                upad, vpad, wdot, b0p, e)
    out = _tc_combine(pmsg, exsum.reshape(NW, n).T, wn)
    return out.reshape(n, 1, D)


# speedup vs baseline: 5.3409x; 1.7240x over previous
import functools

import jax
import jax.numpy as jnp
from jax import lax
from jax.experimental import pallas as pl
from jax.experimental.pallas import tpu as pltpu
from jax.experimental.pallas import tpu_sc as plsc

N_NODES = 10000
D = 128
DH = 32
NPH = D // DH
DE = 16

NC = 2
NS = 16
NW = NC * NS

B = 64
N_PAD = 10240
ROWS_PER_TILE = N_PAD // NS
ROW_CHUNK = 128

_HI = jax.lax.Precision.HIGHEST


def _lrelu(x):
    return jnp.where(x > 0, x, 0.2 * x)


def _tc_node_transform(x, w_att_cat, b_att_cat, w_msg_cat, b_msg_cat,
                       w_wn, b_wn):
    n = x.shape[0]
    rn = 2000
    grid = n // rn

    def body(x_ref, wa_ref, ba_ref, wm_ref, bm_ref, wn_ref, bn_ref,
             a_ref, b_ref, *rest):
        ms_refs = rest[0:NPH]
        md_refs = rest[NPH:2 * NPH]
        own_ref = rest[2 * NPH]
        x_blk = x_ref[...]
        att = jnp.dot(x_blk, wa_ref[...], precision=_HI) + ba_ref[...]
        a_ref[...] = att[:, :D]
        b_ref[...] = att[:, D:]
        msg = jnp.dot(x_blk, wm_ref[...], precision=_HI) + bm_ref[...]
        for q in range(NPH):
            ms_refs[q][...] = msg[:, q * DH:(q + 1) * DH]
            md_refs[q][...] = msg[:, D + q * DH:D + (q + 1) * DH]
        own_ref[...] = jnp.dot(x_blk, wn_ref[...], precision=_HI) + bn_ref[...]

    full = lambda shape: pl.BlockSpec(shape, lambda i: (0,) * len(shape))
    half = pl.BlockSpec((rn, DH), lambda i: (i, 0))
    return pl.pallas_call(
        body,
        grid=(grid,),
        in_specs=[
            pl.BlockSpec((rn, D), lambda i: (i, 0)),
            full((D, 2 * D)), full((1, 2 * D)),
            full((D, 2 * D)), full((1, 2 * D)),
            full((D, D)), full((1, D)),
        ],
        out_specs=(
            [pl.BlockSpec((rn, D), lambda i: (i, 0))] * 2
            + [half] * (2 * NPH)
            + [pl.BlockSpec((rn, D), lambda i: (i, 0))]),
        out_shape=(
            [jax.ShapeDtypeStruct((n, D), jnp.float32)] * 2
            + [jax.ShapeDtypeStruct((n, DH), jnp.float32)] * (2 * NPH)
            + [jax.ShapeDtypeStruct((n, D), jnp.float32)]),
    )(x, w_att_cat, b_att_cat, w_msg_cat, b_msg_cat, w_wn, b_wn)


def _tc_edge_transform(ef4, w_ae, b_ae, wm_blk, bm_q):
    epad4 = ef4.shape[0]
    re = 1280
    grid = epad4 // re

    def body(ef_ref, wa_ref, ba_ref, w0, w1, w2, w3, b0, b1, b2, b3,
             ea0, ea1, ea2, ea3, em0, em1, em2, em3):
        x4 = ef_ref[...]
        ea_refs = [ea0, ea1, ea2, ea3]
        em_refs = [em0, em1, em2, em3]
        wq = [w0, w1, w2, w3]
        bq = [b0, b1, b2, b3]
        for k in range(4):
            ea_refs[k][...] = (
                jnp.dot(x4[:, 16 * k:16 * (k + 1)], wa_ref[...],
                        precision=_HI) + ba_ref[...])
        for q in range(4):
            em_refs[q][...] = jnp.dot(x4, wq[q][...]) + bq[q][...]

    blk = pl.BlockSpec((re, DE * 4), lambda i: (i, 0))
    wfull = pl.BlockSpec((DE, D), lambda i: (0, 0))
    wblk = pl.BlockSpec((DE * 4, D), lambda i: (0, 0))
    bfull = pl.BlockSpec((1, D), lambda i: (0, 0))
    oblk = pl.BlockSpec((re, D), lambda i: (i, 0))
    return pl.pallas_call(
        body,
        grid=(grid,),
        in_specs=[blk, wfull, bfull] + [wblk] * 4 + [bfull] * 4,
        out_specs=[oblk] * 8,
        out_shape=[jax.ShapeDtypeStruct((epad4, D), jnp.float32)] * 8,
    )(ef4, w_ae, b_ae, *wm_blk, *bm_q)


def _sc_edge_pass(ta, tb, ms_q, md_q, ea_k, em_q,
                  upad, vpad, wdot, b0p, n_edges):
    epad = upad.shape[0]
    ept = epad // NW
    nb = ept // B
    nb0 = (2 * nb * 59 // 100) // 4 * 4
    nb1 = 2 * nb - nb0
    assert nb0 % 4 == 0 and nb1 % 4 == 0 and min(nb0, nb1) >= 8
    ept_max = max(nb0, nb1) * B
    mesh = plsc.VectorSubcoreMesh(core_axis_name="c", subcore_axis_name="s")

    @functools.partial(
        pl.kernel,
        mesh=mesh,
        compiler_params=pltpu.CompilerParams(
            needs_layout_passes=False, use_tc_tiling_on_sc=False),
        out_type=[
            jax.ShapeDtypeStruct((NW, 1, N_NODES), jnp.float32),
            jax.ShapeDtypeStruct((NC, NPH, N_PAD, DH), jnp.float32),
        ],
        scratch_types=(
            [pltpu.VMEM((B,), jnp.int32)] * 4
            + [pltpu.VMEM((B,), jnp.int32)] * 4
            + [pltpu.VMEM((B,), jnp.int32)] * 2
            + [pltpu.VMEM((B, D), jnp.float32)] * 6
            + [pltpu.VMEM((B, DH), jnp.float32)] * 4
            + [pltpu.VMEM((B // 4, D), jnp.float32)] * 2
            + [pltpu.VMEM((B, DH), jnp.float32)] * 2
            + [
                pltpu.VMEM((ept_max,), jnp.float32),
                pltpu.VMEM((N_NODES,), jnp.float32),
                pltpu.VMEM((D,), jnp.float32),
                pltpu.VMEM((16,), jnp.float32),
                pltpu.VMEM((ROW_CHUNK, DH), jnp.float32),
                pltpu.VMEM_SHARED((N_PAD, DH), jnp.float32),
            ]
            + [pltpu.SemaphoreType.DMA] * 8
        ),
    )
    def sc_kernel(ta_h, tb_h,
                  ms0_h, ms1_h, ms2_h, ms3_h, md0_h, md1_h, md2_h, md3_h,
                  ea0_h, ea1_h, ea2_h, ea3_h, em0_h, em1_h, em2_h, em3_h,
                  u_h, v_h, w_h, b0_h,
                  exsum_h, pmsg_h,
                  u0, u1, u2, u3, v0, v1, v2, v3, vsc0, vsc1,
                  ga0, ga1, gb0, gb1, ge0, ge1,
                  gms0, gms1, gmd0, gmd1, gem0, gem1, mv0, mv1,
                  ex_all, tbl, w_v, b0_v, zbuf, shacc,
                  si0, si1, si2, si3, sg0, sg1, ss0, ss1):
        ms_hs = [ms0_h, ms1_h, ms2_h, ms3_h]
        md_hs = [md0_h, md1_h, md2_h, md3_h]
        em_hs = [em0_h, em1_h, em2_h, em3_h]
        ea_hs = [ea0_h, ea1_h, ea2_h, ea3_h]
        u_vs = [u0, u1, u2, u3]
        v_vs = [v0, v1, v2, v3]
        vsc = [vsc0, vsc1]
        g_a = [ga0, ga1]
        g_b = [gb0, gb1]
        g_e = [ge0, ge1]
        g_ms = [gms0, gms1]
        g_md = [gmd0, gmd1]
        g_em = [gem0, gem1]
        msg_v = [mv0, mv1]
        sem_i = [si0, si1, si2, si3]
        sem_g = [sg0, sg1]
        sem_sc = [ss0, ss1]

        c = lax.axis_index("c")
        s = lax.axis_index("s")
        wid = c * NS + s

        pltpu.sync_copy(w_h, w_v)
        pltpu.sync_copy(b0_h, b0_v)
        wch = [w_v[pl.ds(i * 16, 16)] for i in range(8)]
        b0s = b0_v[...][0]
        zero16 = jnp.zeros((16,), jnp.float32)
        iota16 = lax.iota(jnp.int32, 16)

        def zt(i, carry):
            tbl[pl.ds(i * 16, 16)] = zero16
            return carry
        lax.fori_loop(0, N_NODES // 16, zt, 0)

        def zz(i, carry):
            for ci in range(DH // 16):
                zbuf[i, pl.ds(ci * 16, 16)] = zero16
            return carry
        lax.fori_loop(0, ROW_CHUNK, zz, 0)

        def zero_shacc(j, carry):
            pltpu.sync_copy(
                zbuf, shacc.at[pl.ds(s * ROWS_PER_TILE + j * ROW_CHUNK,
                                     ROW_CHUNK)])
            return carry
        lax.fori_loop(0, ROWS_PER_TILE // ROW_CHUNK, zero_shacc, 0)

        nbc = jnp.where(c == 0, nb0, nb1)
        ebase0 = jnp.where(c == 0, s * (nb0 * B),
                           NS * nb0 * B + s * (nb1 * B))
        ebase04 = ebase0 // 4

        def idx_load(ib, k):
            base = pl.multiple_of(ebase0 + ib * B, B)
            pltpu.async_copy(u_h.at[pl.ds(base, B)], u_vs[k], sem_i[k])
            pltpu.async_copy(v_h.at[pl.ds(base, B)], v_vs[k], sem_i[k])

        def idx_wait(ib, k):
            base = pl.multiple_of(ebase0 + ib * B, B)
            pltpu.make_async_copy(
                u_h.at[pl.ds(base, B)], u_vs[k], sem_i[k]).wait()
            pltpu.make_async_copy(
                v_h.at[pl.ds(base, B)], v_vs[k], sem_i[k]).wait()

        B4 = B // 4

        def run_pipeline(src_h, dst_h, edg_cp, gsrc, gdst, compute):
            def gstart(ib, gs, k):
                pltpu.async_copy(src_h.at[u_vs[k]], gsrc[gs], sem_g[gs])
                pltpu.async_copy(dst_h.at[v_vs[k]], gdst[gs], sem_g[gs])
                edg_cp(ib, gs, False)

            def gwait(ib, gs, k):
                pltpu.make_async_copy(
                    src_h.at[u_vs[k]], gsrc[gs], sem_g[gs]).wait()
                pltpu.make_async_copy(
                    dst_h.at[v_vs[k]], gdst[gs], sem_g[gs]).wait()
                edg_cp(ib, gs, True)

            for k in range(4):
                idx_load(k, k)
            idx_wait(0, 0)
            gstart(0, 0, 0)
            idx_wait(1, 1)
            gstart(1, 1, 1)

            def quad(h, carry):
                ib0 = h * 4
                for j in range(4):
                    ib = ib0 + j
                    gs = j % 2
                    gwait(ib, gs, j)
                    compute(ib, gs, j)

                    @pl.when(ib + 2 < nbc)
                    def _():
                        idx_wait(ib + 2, (j + 2) % 4)
                        gstart(ib + 2, gs, (j + 2) % 4)

                    @pl.when(ib + 4 < nbc)
                    def _():
                        idx_load(ib + 4, j)
                return carry
            lax.fori_loop(0, nbc // 4, quad, 0)

        def ea_cp(ib, gs, is_wait):
            base4 = pl.multiple_of(ebase04 + ib * B4, B4)
            for k in range(4):
                args = (ea_hs[k].at[pl.ds(base4, B4)],
                        g_e[gs].at[pl.ds(k * B4, B4)], sem_g[gs])
                if is_wait:
                    pltpu.make_async_copy(*args).wait()
                else:
                    pltpu.async_copy(*args)

        def att_compute(ib, gs, k):
            base = pl.multiple_of(ebase0 + ib * B, B)
            ga, gb, ge = g_a[gs], g_b[gs], g_e[gs]
            vk = v_vs[k]

            def att_grp(g, carry2):
                e0 = g * 16
                lgv = zero16
                for j in range(16):
                    e = e0 + j
                    re4 = (j % 4) * B4 + 4 * g + j // 4
                    acc0 = zero16
                    acc1 = zero16
                    for ci in range(8):
                        sl = pl.ds(ci * 16, 16)
                        q = ga[e, sl] + gb[e, sl] + ge[re4, sl]
                        q = jnp.maximum(q, 0.2 * q)
                        if ci % 2 == 0:
                            acc0 = acc0 + wch[ci] * q
                        else:
                            acc1 = acc1 + wch[ci] * q
                    lg = jnp.sum(acc0 + acc1)
                    lgv = jnp.where(iota16 == j, lg, lgv)
                eid = base + e0 + iota16
                ex = jnp.where(eid < n_edges, jnp.exp(lgv + b0s), 0.0)
                ex_all[pl.ds(ib * B + e0, 16)] = ex
                plsc.addupdate_scatter(tbl, [vk[pl.ds(e0, 16)]], ex)
                return carry2
            lax.fori_loop(0, B // 16, att_grp, 0)

        run_pipeline(ta_h, tb_h, ea_cp, g_a, g_b, att_compute)
        pltpu.sync_copy(tbl, exsum_h.at[wid, 0])

        for ph in range(NPH):
            ms_h, md_h, em_h = ms_hs[ph], md_hs[ph], em_hs[ph]
            if ph > 0:
                lax.fori_loop(0, ROWS_PER_TILE // ROW_CHUNK, zero_shacc, 0)
            plsc.subcore_barrier()

            def sc_wait(gs):
                pltpu.make_async_copy(
                    msg_v[gs], shacc.at[vsc[gs]], sem_sc[gs]).wait()

            def em_cp(ib, gs, is_wait, em_h=em_h):
                base4 = pl.multiple_of(ebase04 + ib * B4, B4)
                args = (em_h.at[pl.ds(base4, B4)], g_em[gs], sem_g[gs])
                if is_wait:
                    pltpu.make_async_copy(*args).wait()
                else:
                    pltpu.async_copy(*args)

            def msg_compute(ib, gs, k):
                @pl.when(ib >= 2)
                def _():
                    sc_wait(gs)

                gms, gmd, gem = g_ms[gs], g_md[gs], g_em[gs]
                mv = msg_v[gs]
                vk = v_vs[k]
                vs = vsc[gs]

                def msg_grp(g, carry2):
                    e0 = g * 16
                    exg = ex_all[pl.ds(ib * B + e0, 16)]
                    for j in range(16):
                        e = e0 + j
                        re4 = 4 * g + j // 4
                        cb = (j % 4) * DH
                        exv = jnp.full((16,), exg[j])
                        for ci in range(DH // 16):
                            sl = pl.ds(ci * 16, 16)
                            t = (gms[e, sl] + gmd[e, sl]
                                 + gem[re4, pl.ds(cb + ci * 16, 16)])
                            mv[e, sl] = jnp.maximum(t, 0.2 * t) * exv
                    vs[pl.ds(e0, 16)] = vk[pl.ds(e0, 16)]
                    return carry2
                lax.fori_loop(0, B // 16, msg_grp, 0)
                pltpu.async_copy(mv, shacc.at[vs], sem_sc[gs], add=True)

            run_pipeline(ms_h, md_h, em_cp, g_ms, g_md, msg_compute)
            sc_wait(0)
            sc_wait(1)
            plsc.subcore_barrier()

            def wo(j, carry):
                r0 = s * ROWS_PER_TILE + j * ROW_CHUNK
                pltpu.sync_copy(shacc.at[pl.ds(r0, ROW_CHUNK)], zbuf)
                pltpu.sync_copy(zbuf, pmsg_h.at[c, ph, pl.ds(r0, ROW_CHUNK)])
                return carry
            lax.fori_loop(0, ROWS_PER_TILE // ROW_CHUNK, wo, 0)
            plsc.subcore_barrier()
            lax.fori_loop(0, ROW_CHUNK, zz, 0)

    return sc_kernel(ta, tb, *ms_q, *md_q, *ea_k, *em_q,
                     upad, vpad, wdot, b0p)


def _tc_combine(pmsg, exsum, wn):
    n = wn.shape[0]
    rn = 2000
    grid = n // rn

    def body(p_ref, s_ref, wn_ref, out_ref):
        p = jnp.concatenate(
            [p_ref[0, q] + p_ref[1, q] for q in range(NPH)], axis=1)
        ssum = jnp.sum(s_ref[...], axis=1)
        r = jnp.where(ssum > 0, 1.0 / ssum, 0.0)
        out_ref[...] = _lrelu(p * r[:, None] + wn_ref[...])

    return pl.pallas_call(
        body,
        grid=(grid,),
        in_specs=[
            pl.BlockSpec((NC, NPH, rn, DH), lambda i: (0, 0, i, 0)),
            pl.BlockSpec((rn, NW), lambda i: (i, 0)),
            pl.BlockSpec((rn, D), lambda i: (i, 0)),
        ],
        out_specs=pl.BlockSpec((rn, D), lambda i: (i, 0)),
        out_shape=jax.ShapeDtypeStruct((n, D), jnp.float32),
    )(pmsg, exsum, wn)


def kernel(node_feats, edge_feats, edge_index,
           W_attn_src, b_attn_src, W_attn_dst, b_attn_dst,
           W_attn_edg, b_attn_edg, W_attn_dot, b_attn_dot,
           W_msg_src, b_msg_src, W_msg_dst, b_msg_dst,
           W_msg_edg, b_msg_edg, W_wgt_n, b_wgt_n):
    n = node_feats.shape[0]
    e = edge_feats.shape[0]
    x = node_feats.reshape(n, D)

    ept = ((e + NW * 4 * B - 1) // (NW * 4 * B)) * 4 * B
    epad = ept * NW
    pad = epad - e
    upad = jnp.pad(edge_index[0], (0, pad))
    vpad = jnp.pad(edge_index[1], (0, pad))
    ef4p = jnp.pad(edge_feats.reshape(e // 4, 4 * DE),
                   ((0, (epad - e) // 4), (0, 0)))

    w_att_cat = jnp.concatenate([W_attn_src, W_attn_dst], axis=0).T
    b_att_cat = jnp.concatenate([b_attn_src, b_attn_dst]).reshape(1, 2 * D)
    w_msg_cat = jnp.concatenate([W_msg_src, W_msg_dst], axis=0).T
    b_msg_cat = jnp.concatenate([b_msg_src, b_msg_dst]).reshape(1, 2 * D)
    wme_t = W_msg_edg.T
    wm_blk = []
    bm_q = []
    for q in range(NPH):
        wq = jnp.zeros((4 * DE, D), jnp.float32)
        for k in range(4):
            wq = wq.at[DE * k:DE * (k + 1),
                       DH * k:DH * (k + 1)].set(wme_t[:, DH * q:DH * (q + 1)])
        wm_blk.append(wq)
        bm_q.append(jnp.tile(b_msg_edg[DH * q:DH * (q + 1)], 4).reshape(1, D))

    outs = _tc_node_transform(
        x, w_att_cat, b_att_cat, w_msg_cat, b_msg_cat,
        W_wgt_n.T, b_wgt_n.reshape(1, D))
    ta, tb = outs[0], outs[1]
    ms_q = outs[2:2 + NPH]
    md_q = outs[2 + NPH:2 + 2 * NPH]
    wn = outs[2 + 2 * NPH]
    eaem = _tc_edge_transform(ef4p, W_attn_edg.T,
                              b_attn_edg.reshape(1, D), wm_blk, bm_q)
    ea_k, em_q = eaem[:4], eaem[4:]

    wdot = W_attn_dot.reshape(D)
    b0p = jnp.pad(b_attn_dot, (0, 15))

    exsum, pmsg = _sc_edge_pass(ta, tb, ms_q, md_q, ea_k, em_q,
                                upad, vpad, wdot, b0p, e)
    out = _tc_combine(pmsg, exsum.reshape(NW, n).T, wn)
    return out.reshape(n, 1, D)


# speedup vs baseline: 5.3780x; 1.0069x over previous
import functools

import jax
import jax.numpy as jnp
from jax import lax
from jax.experimental import pallas as pl
from jax.experimental.pallas import tpu as pltpu
from jax.experimental.pallas import tpu_sc as plsc

N_NODES = 10000
D = 128
DH = 32
NPH = D // DH
DE = 16

NC = 2
NS = 16
NW = NC * NS

B = 64
N_PAD = 10240
ROWS_PER_TILE = N_PAD // NS
ROW_CHUNK = 128

_HI = jax.lax.Precision.HIGHEST


def _lrelu(x):
    return jnp.where(x > 0, x, 0.2 * x)


def _tc_node_transform(x, w_att_cat, b_att_cat, w_msg_cat, b_msg_cat,
                       w_wn, b_wn):
    n = x.shape[0]
    rn = 2000
    grid = n // rn

    def body(x_ref, wa_ref, ba_ref, wm_ref, bm_ref, wn_ref, bn_ref,
             a_ref, b_ref, *rest):
        ms_refs = rest[0:NPH]
        md_refs = rest[NPH:2 * NPH]
        own_ref = rest[2 * NPH]
        x_blk = x_ref[...]
        att = jnp.dot(x_blk, wa_ref[...], precision=_HI) + ba_ref[...]
        a_ref[...] = att[:, :D]
        b_ref[...] = att[:, D:]
        msg = jnp.dot(x_blk, wm_ref[...], precision=_HI) + bm_ref[...]
        for q in range(NPH):
            ms_refs[q][...] = msg[:, q * DH:(q + 1) * DH]
            md_refs[q][...] = msg[:, D + q * DH:D + (q + 1) * DH]
        own_ref[...] = jnp.dot(x_blk, wn_ref[...], precision=_HI) + bn_ref[...]

    full = lambda shape: pl.BlockSpec(shape, lambda i: (0,) * len(shape))
    half = pl.BlockSpec((rn, DH), lambda i: (i, 0))
    return pl.pallas_call(
        body,
        grid=(grid,),
        in_specs=[
            pl.BlockSpec((rn, D), lambda i: (i, 0)),
            full((D, 2 * D)), full((1, 2 * D)),
            full((D, 2 * D)), full((1, 2 * D)),
            full((D, D)), full((1, D)),
        ],
        out_specs=(
            [pl.BlockSpec((rn, D), lambda i: (i, 0))] * 2
            + [half] * (2 * NPH)
            + [pl.BlockSpec((rn, D), lambda i: (i, 0))]),
        out_shape=(
            [jax.ShapeDtypeStruct((n, D), jnp.float32)] * 2
            + [jax.ShapeDtypeStruct((n, DH), jnp.float32)] * (2 * NPH)
            + [jax.ShapeDtypeStruct((n, D), jnp.float32)]),
    )(x, w_att_cat, b_att_cat, w_msg_cat, b_msg_cat, w_wn, b_wn)


def _tc_edge_transform(ef4, w_ae, b_ae, wm_blk, bm_q):
    epad4 = ef4.shape[0]
    re = 1280
    grid = epad4 // re

    def body(ef_ref, wa_ref, ba_ref, w0, w1, w2, w3, b0, b1, b2, b3,
             ea0, ea1, ea2, ea3, em0, em1, em2, em3):
        x4 = ef_ref[...]
        ea_refs = [ea0, ea1, ea2, ea3]
        em_refs = [em0, em1, em2, em3]
        wq = [w0, w1, w2, w3]
        bq = [b0, b1, b2, b3]
        for k in range(4):
            ea_refs[k][...] = (
                jnp.dot(x4[:, 16 * k:16 * (k + 1)], wa_ref[...],
                        precision=_HI) + ba_ref[...])
        for q in range(4):
            em_refs[q][...] = jnp.dot(x4, wq[q][...]) + bq[q][...]

    blk = pl.BlockSpec((re, DE * 4), lambda i: (i, 0))
    wfull = pl.BlockSpec((DE, D), lambda i: (0, 0))
    wblk = pl.BlockSpec((DE * 4, D), lambda i: (0, 0))
    bfull = pl.BlockSpec((1, D), lambda i: (0, 0))
    oblk = pl.BlockSpec((re, D), lambda i: (i, 0))
    return pl.pallas_call(
        body,
        grid=(grid,),
        in_specs=[blk, wfull, bfull] + [wblk] * 4 + [bfull] * 4,
        out_specs=[oblk] * 8,
        out_shape=[jax.ShapeDtypeStruct((epad4, D), jnp.float32)] * 8,
    )(ef4, w_ae, b_ae, *wm_blk, *bm_q)


def _sc_edge_pass(ta, tb, ms_q, md_q, ea_k, em_q,
                  upad, vpad, wdot, b0p, n_edges):
    epad = upad.shape[0]
    ept = epad // NW
    nb = ept // B
    nb0 = (2 * nb * 3 // 5) // 4 * 4
    nb1 = 2 * nb - nb0
    assert nb0 % 4 == 0 and nb1 % 4 == 0 and min(nb0, nb1) >= 8
    ept_max = max(nb0, nb1) * B
    mesh = plsc.VectorSubcoreMesh(core_axis_name="c", subcore_axis_name="s")

    @functools.partial(
        pl.kernel,
        mesh=mesh,
        compiler_params=pltpu.CompilerParams(
            needs_layout_passes=False, use_tc_tiling_on_sc=False),
        out_type=[
            jax.ShapeDtypeStruct((NW, 1, N_NODES), jnp.float32),
            jax.ShapeDtypeStruct((NC, NPH, N_PAD, DH), jnp.float32),
        ],
        scratch_types=(
            [pltpu.VMEM((B,), jnp.int32)] * 4
            + [pltpu.VMEM((B,), jnp.int32)] * 4
            + [pltpu.VMEM((B,), jnp.int32)] * 2
            + [pltpu.VMEM((B, D), jnp.float32)] * 6
            + [pltpu.VMEM((B, DH), jnp.float32)] * 4
            + [pltpu.VMEM((B // 4, D), jnp.float32)] * 2
            + [pltpu.VMEM((B, DH), jnp.float32)] * 2
            + [
                pltpu.VMEM((ept_max,), jnp.float32),
                pltpu.VMEM((N_NODES,), jnp.float32),
                pltpu.VMEM((D,), jnp.float32),
                pltpu.VMEM((16,), jnp.float32),
                pltpu.VMEM((ROW_CHUNK, DH), jnp.float32),
                pltpu.VMEM_SHARED((N_PAD, DH), jnp.float32),
            ]
            + [pltpu.SemaphoreType.DMA] * 8
        ),
    )
    def sc_kernel(ta_h, tb_h,
                  ms0_h, ms1_h, ms2_h, ms3_h, md0_h, md1_h, md2_h, md3_h,
                  ea0_h, ea1_h, ea2_h, ea3_h, em0_h, em1_h, em2_h, em3_h,
                  u_h, v_h, w_h, b0_h,
                  exsum_h, pmsg_h,
                  u0, u1, u2, u3, v0, v1, v2, v3, vsc0, vsc1,
                  ga0, ga1, gb0, gb1, ge0, ge1,
                  gms0, gms1, gmd0, gmd1, gem0, gem1, mv0, mv1,
                  ex_all, tbl, w_v, b0_v, zbuf, shacc,
                  si0, si1, si2, si3, sg0, sg1, ss0, ss1):
        ms_hs = [ms0_h, ms1_h, ms2_h, ms3_h]
        md_hs = [md0_h, md1_h, md2_h, md3_h]
        em_hs = [em0_h, em1_h, em2_h, em3_h]
        ea_hs = [ea0_h, ea1_h, ea2_h, ea3_h]
        u_vs = [u0, u1, u2, u3]
        v_vs = [v0, v1, v2, v3]
        vsc = [vsc0, vsc1]
        g_a = [ga0, ga1]
        g_b = [gb0, gb1]
        g_e = [ge0, ge1]
        g_ms = [gms0, gms1]
        g_md = [gmd0, gmd1]
        g_em = [gem0, gem1]
        msg_v = [mv0, mv1]
        sem_i = [si0, si1, si2, si3]
        sem_g = [sg0, sg1]
        sem_sc = [ss0, ss1]

        c = lax.axis_index("c")
        s = lax.axis_index("s")
        wid = c * NS + s

        pltpu.sync_copy(w_h, w_v)
        pltpu.sync_copy(b0_h, b0_v)
        wch = [w_v[pl.ds(i * 16, 16)] for i in range(8)]
        b0s = b0_v[...][0]
        zero16 = jnp.zeros((16,), jnp.float32)
        iota16 = lax.iota(jnp.int32, 16)

        def zt(i, carry):
            tbl[pl.ds(i * 16, 16)] = zero16
            return carry
        lax.fori_loop(0, N_NODES // 16, zt, 0)

        def zz(i, carry):
            for ci in range(DH // 16):
                zbuf[i, pl.ds(ci * 16, 16)] = zero16
            return carry
        lax.fori_loop(0, ROW_CHUNK, zz, 0)

        def zero_shacc(j, carry):
            pltpu.sync_copy(
                zbuf, shacc.at[pl.ds(s * ROWS_PER_TILE + j * ROW_CHUNK,
                                     ROW_CHUNK)])
            return carry
        lax.fori_loop(0, ROWS_PER_TILE // ROW_CHUNK, zero_shacc, 0)

        nbc = jnp.where(c == 0, nb0, nb1)
        ebase0 = jnp.where(c == 0, s * (nb0 * B),
                           NS * nb0 * B + s * (nb1 * B))
        ebase04 = ebase0 // 4

        def idx_load(ib, k):
            base = pl.multiple_of(ebase0 + ib * B, B)
            pltpu.async_copy(u_h.at[pl.ds(base, B)], u_vs[k], sem_i[k])
            pltpu.async_copy(v_h.at[pl.ds(base, B)], v_vs[k], sem_i[k])

        def idx_wait(ib, k):
            base = pl.multiple_of(ebase0 + ib * B, B)
            pltpu.make_async_copy(
                u_h.at[pl.ds(base, B)], u_vs[k], sem_i[k]).wait()
            pltpu.make_async_copy(
                v_h.at[pl.ds(base, B)], v_vs[k], sem_i[k]).wait()

        B4 = B // 4

        def run_pipeline(src_h, dst_h, edg_cp, gsrc, gdst, compute):
            def gstart(ib, gs, k):
                pltpu.async_copy(src_h.at[u_vs[k]], gsrc[gs], sem_g[gs])
                pltpu.async_copy(dst_h.at[v_vs[k]], gdst[gs], sem_g[gs])
                edg_cp(ib, gs, False)

            def gwait(ib, gs, k):
                pltpu.make_async_copy(
                    src_h.at[u_vs[k]], gsrc[gs], sem_g[gs]).wait()
                pltpu.make_async_copy(
                    dst_h.at[v_vs[k]], gdst[gs], sem_g[gs]).wait()
                edg_cp(ib, gs, True)

            for k in range(4):
                idx_load(k, k)
            idx_wait(0, 0)
            gstart(0, 0, 0)
            idx_wait(1, 1)
            gstart(1, 1, 1)

            def quad(h, carry):
                ib0 = h * 4
                for j in range(4):
                    ib = ib0 + j
                    gs = j % 2
                    gwait(ib, gs, j)
                    compute(ib, gs, j)

                    @pl.when(ib + 2 < nbc)
                    def _():
                        idx_wait(ib + 2, (j + 2) % 4)
                        gstart(ib + 2, gs, (j + 2) % 4)

                    @pl.when(ib + 4 < nbc)
                    def _():
                        idx_load(ib + 4, j)
                return carry
            lax.fori_loop(0, nbc // 4, quad, 0)

        def ea_cp(ib, gs, is_wait):
            base4 = pl.multiple_of(ebase04 + ib * B4, B4)
            for k in range(4):
                args = (ea_hs[k].at[pl.ds(base4, B4)],
                        g_e[gs].at[pl.ds(k * B4, B4)], sem_g[gs])
                if is_wait:
                    pltpu.make_async_copy(*args).wait()
                else:
                    pltpu.async_copy(*args)

        def att_compute(ib, gs, k):
            base = pl.multiple_of(ebase0 + ib * B, B)
            ga, gb, ge = g_a[gs], g_b[gs], g_e[gs]
            vk = v_vs[k]

            def att_grp(g, carry2):
                e0 = g * 16
                lgv = zero16
                for j in range(16):
                    e = e0 + j
                    re4 = (j % 4) * B4 + 4 * g + j // 4
                    acc0 = zero16
                    acc1 = zero16
                    for ci in range(8):
                        sl = pl.ds(ci * 16, 16)
                        q = ga[e, sl] + gb[e, sl] + ge[re4, sl]
                        q = jnp.maximum(q, 0.2 * q)
                        if ci % 2 == 0:
                            acc0 = acc0 + wch[ci] * q
                        else:
                            acc1 = acc1 + wch[ci] * q
                    lg = jnp.sum(acc0 + acc1)
                    lgv = jnp.where(iota16 == j, lg, lgv)
                eid = base + e0 + iota16
                ex = jnp.where(eid < n_edges, jnp.exp(lgv + b0s), 0.0)
                ex_all[pl.ds(ib * B + e0, 16)] = ex
                plsc.addupdate_scatter(tbl, [vk[pl.ds(e0, 16)]], ex)
                return carry2
            lax.fori_loop(0, B // 16, att_grp, 0)

        run_pipeline(ta_h, tb_h, ea_cp, g_a, g_b, att_compute)
        pltpu.sync_copy(tbl, exsum_h.at[wid, 0])

        for ph in range(NPH):
            ms_h, md_h, em_h = ms_hs[ph], md_hs[ph], em_hs[ph]
            if ph > 0:
                lax.fori_loop(0, ROWS_PER_TILE // ROW_CHUNK, zero_shacc, 0)
            plsc.subcore_barrier()

            def sc_wait(gs):
                pltpu.make_async_copy(
                    msg_v[gs], shacc.at[vsc[gs]], sem_sc[gs]).wait()

            def em_cp(ib, gs, is_wait, em_h=em_h):
                base4 = pl.multiple_of(ebase04 + ib * B4, B4)
                args = (em_h.at[pl.ds(base4, B4)], g_em[gs], sem_g[gs])
                if is_wait:
                    pltpu.make_async_copy(*args).wait()
                else:
                    pltpu.async_copy(*args)

            def msg_compute(ib, gs, k):
                @pl.when(ib >= 2)
                def _():
                    sc_wait(gs)

                gms, gmd, gem = g_ms[gs], g_md[gs], g_em[gs]
                mv = msg_v[gs]
                vk = v_vs[k]
                vs = vsc[gs]

                def msg_grp(g, carry2):
                    e0 = g * 16
                    exg = ex_all[pl.ds(ib * B + e0, 16)]
                    for j in range(16):
                        e = e0 + j
                        re4 = 4 * g + j // 4
                        cb = (j % 4) * DH
                        exv = jnp.full((16,), exg[j])
                        for ci in range(DH // 16):
                            sl = pl.ds(ci * 16, 16)
                            t = (gms[e, sl] + gmd[e, sl]
                                 + gem[re4, pl.ds(cb + ci * 16, 16)])
                            mv[e, sl] = jnp.maximum(t, 0.2 * t) * exv
                    vs[pl.ds(e0, 16)] = vk[pl.ds(e0, 16)]
                    return carry2
                lax.fori_loop(0, B // 16, msg_grp, 0)
                pltpu.async_copy(mv, shacc.at[vs], sem_sc[gs], add=True)

            run_pipeline(ms_h, md_h, em_cp, g_ms, g_md, msg_compute)
            sc_wait(0)
            sc_wait(1)
            plsc.subcore_barrier()

            def wo(j, carry):
                r0 = s * ROWS_PER_TILE + j * ROW_CHUNK
                pltpu.sync_copy(shacc.at[pl.ds(r0, ROW_CHUNK)], zbuf)
                pltpu.sync_copy(zbuf, pmsg_h.at[c, ph, pl.ds(r0, ROW_CHUNK)])
                return carry
            lax.fori_loop(0, ROWS_PER_TILE // ROW_CHUNK, wo, 0)
            plsc.subcore_barrier()
            lax.fori_loop(0, ROW_CHUNK, zz, 0)

    return sc_kernel(ta, tb, *ms_q, *md_q, *ea_k, *em_q,
                     upad, vpad, wdot, b0p)


def _tc_combine(pmsg, exsum, wn):
    n = wn.shape[0]
    rn = 2000
    grid = n // rn

    def body(p_ref, s_ref, wn_ref, out_ref):
        p = jnp.concatenate(
            [p_ref[0, q] + p_ref[1, q] for q in range(NPH)], axis=1)
        ssum = jnp.sum(s_ref[...], axis=1)
        r = jnp.where(ssum > 0, 1.0 / ssum, 0.0)
        out_ref[...] = _lrelu(p * r[:, None] + wn_ref[...])

    return pl.pallas_call(
        body,
        grid=(grid,),
        in_specs=[
            pl.BlockSpec((NC, NPH, rn, DH), lambda i: (0, 0, i, 0)),
            pl.BlockSpec((rn, NW), lambda i: (i, 0)),
            pl.BlockSpec((rn, D), lambda i: (i, 0)),
        ],
        out_specs=pl.BlockSpec((rn, D), lambda i: (i, 0)),
        out_shape=jax.ShapeDtypeStruct((n, D), jnp.float32),
    )(pmsg, exsum, wn)


def kernel(node_feats, edge_feats, edge_index,
           W_attn_src, b_attn_src, W_attn_dst, b_attn_dst,
           W_attn_edg, b_attn_edg, W_attn_dot, b_attn_dot,
           W_msg_src, b_msg_src, W_msg_dst, b_msg_dst,
           W_msg_edg, b_msg_edg, W_wgt_n, b_wgt_n):
    n = node_feats.shape[0]
    e = edge_feats.shape[0]
    x = node_feats.reshape(n, D)

    ept = ((e + NW * 4 * B - 1) // (NW * 4 * B)) * 4 * B
    epad = ept * NW
    pad = epad - e
    upad = jnp.pad(edge_index[0], (0, pad))
    vpad = jnp.pad(edge_index[1], (0, pad))
    ef4p = jnp.pad(edge_feats.reshape(e // 4, 4 * DE),
                   ((0, (epad - e) // 4), (0, 0)))

    w_att_cat = jnp.concatenate([W_attn_src, W_attn_dst], axis=0).T
    b_att_cat = jnp.concatenate([b_attn_src, b_attn_dst]).reshape(1, 2 * D)
    w_msg_cat = jnp.concatenate([W_msg_src, W_msg_dst], axis=0).T
    b_msg_cat = jnp.concatenate([b_msg_src, b_msg_dst]).reshape(1, 2 * D)
    wme_t = W_msg_edg.T
    wm_blk = []
    bm_q = []
    for q in range(NPH):
        wq = jnp.zeros((4 * DE, D), jnp.float32)
        for k in range(4):
            wq = wq.at[DE * k:DE * (k + 1),
                       DH * k:DH * (k + 1)].set(wme_t[:, DH * q:DH * (q + 1)])
        wm_blk.append(wq)
        bm_q.append(jnp.tile(b_msg_edg[DH * q:DH * (q + 1)], 4).reshape(1, D))

    outs = _tc_node_transform(
        x, w_att_cat, b_att_cat, w_msg_cat, b_msg_cat,
        W_wgt_n.T, b_wgt_n.reshape(1, D))
    ta, tb = outs[0], outs[1]
    ms_q = outs[2:2 + NPH]
    md_q = outs[2 + NPH:2 + 2 * NPH]
    wn = outs[2 + 2 * NPH]
    eaem = _tc_edge_transform(ef4p, W_attn_edg.T,
                              b_attn_edg.reshape(1, D), wm_blk, bm_q)
    ea_k, em_q = eaem[:4], eaem[4:]

    wdot = W_attn_dot.reshape(D)
    b0p = jnp.pad(b_attn_dot, (0, 15))

    exsum, pmsg = _sc_edge_pass(ta, tb, ms_q, md_q, ea_k, em_q,
                                upad, vpad, wdot, b0p, e)
    out = _tc_combine(pmsg, exsum.reshape(NW, n).T, wn)
    return out.reshape(n, 1, D)
